# Initial kernel scaffold; baseline (speedup 1.0000x reference)
import jax
import jax.numpy as jnp
from jax.experimental import pallas as pl


def kernel(x, edge_index, params):
    raise NotImplementedError("write your pallas kernel here")


# speedup vs baseline: 11.7066x; 11.7066x over previous
import functools

import jax
import jax.numpy as jnp
import numpy as np
from jax import lax
from jax.experimental import pallas as pl
from jax.experimental.pallas import tpu as pltpu
from jax.experimental.pallas import tpu_sc as plsc

N = 50000
E = 800000
IC, H, OC = 8, 96, 2
BD = H // 3
HEADS, DH = 4, H // 4
SQRT_B = np.sqrt(BD).astype(np.float32)

NW = 32
E_PAD = 819200
EW = E_PAD // NW
NCH = EW // 128
NPAD = 50176
NPS = NPAD // 16

RBN = 2000
GBN = N // RBN
RBE = 8192
GBE = E_PAD // RBE
RBE2 = 4096
GBE2 = E_PAD // RBE2

@functools.lru_cache(maxsize=None)
def _mesh():
    return plsc.VectorSubcoreMesh(core_axis_name="c", subcore_axis_name="s")


_INV_SQRT2 = np.float32(1.0 / np.sqrt(2.0))


def _gelu(x):
    return 0.5 * x * (1.0 + lax.erf(x * _INV_SQRT2))


def _rows(rb, d):
    return pl.BlockSpec((rb, d), lambda i: (i, 0))


def _part(rb, d):
    return pl.BlockSpec((2, rb, d), lambda i: (0, i, 0))


def _full(shape):
    return pl.BlockSpec(shape, lambda i: tuple(0 for _ in shape))


def _smem(shape):
    return pl.BlockSpec(shape, lambda i: tuple(0 for _ in shape),
                        memory_space=pltpu.SMEM)


def _f32(shape):
    return jax.ShapeDtypeStruct(shape, jnp.float32)


@functools.lru_cache(maxsize=None)
def _sc_segsum_gather_fn(D):
    @functools.partial(
        pl.kernel,
        out_type=_f32((2 * NPAD, D)),
        mesh=_mesh(),
        compiler_params=pltpu.CompilerParams(use_tc_tiling_on_sc=False),
        scratch_types=[
            pltpu.VMEM((128,), jnp.int32),
            pltpu.VMEM((128,), jnp.int32),
            pltpu.VMEM((128, D), jnp.float32),
            pltpu.VMEM_SHARED((NPAD, D), jnp.float32),
            pltpu.SemaphoreType.DMA,
        ],
    )
    def k(u_hbm, ig_hbm, is_hbm, z_hbm, out_hbm, ig_v, is_v, buf, acc, sem):
        c = lax.axis_index("c")
        s = lax.axis_index("s")
        wid = s * 2 + c
        pltpu.sync_copy(z_hbm, acc.at[pl.ds(s * NPS, NPS)])
        plsc.subcore_barrier()
        base = wid * EW

        def body(j, carry):
            off = base + j * 128
            pltpu.sync_copy(ig_hbm.at[pl.ds(off, 128)], ig_v)
            pltpu.sync_copy(is_hbm.at[pl.ds(off, 128)], is_v)
            pltpu.async_copy(u_hbm.at[ig_v], buf, sem).wait()
            pltpu.sync_copy(buf, acc.at[is_v], add=True)
            return carry

        lax.fori_loop(0, NCH, body, 0)
        plsc.subcore_barrier()
        pltpu.sync_copy(acc.at[pl.ds(s * NPS, NPS)],
                        out_hbm.at[pl.ds(c * NPAD + s * NPS, NPS)])

    return k


def _sc_segsum_gather(U, ig, is_, D):
    z = jnp.zeros((NPS, D), jnp.float32)
    out = _sc_segsum_gather_fn(D)(U, ig, is_, z)
    return out.reshape(2, NPAD, D)


@functools.lru_cache(maxsize=None)
def _sc_segsum_linear_fn(D):
    @functools.partial(
        pl.kernel,
        out_type=_f32((2 * NPAD, D)),
        mesh=_mesh(),
        compiler_params=pltpu.CompilerParams(use_tc_tiling_on_sc=False),
        scratch_types=[
            pltpu.VMEM((128,), jnp.int32),
            pltpu.VMEM((128, D), jnp.float32),
            pltpu.VMEM_SHARED((NPAD, D), jnp.float32),
        ],
    )
    def k(v_hbm, is_hbm, z_hbm, out_hbm, is_v, buf, acc):
        c = lax.axis_index("c")
        s = lax.axis_index("s")
        wid = s * 2 + c
        pltpu.sync_copy(z_hbm, acc.at[pl.ds(s * NPS, NPS)])
        plsc.subcore_barrier()
        base = wid * EW

        def body(j, carry):
            off = base + j * 128
            pltpu.sync_copy(is_hbm.at[pl.ds(off, 128)], is_v)
            pltpu.sync_copy(v_hbm.at[pl.ds(off, 128)], buf)
            pltpu.sync_copy(buf, acc.at[is_v], add=True)
            return carry

        lax.fori_loop(0, NCH, body, 0)
        plsc.subcore_barrier()
        pltpu.sync_copy(acc.at[pl.ds(s * NPS, NPS)],
                        out_hbm.at[pl.ds(c * NPAD + s * NPS, NPS)])

    return k


def _sc_segsum_linear(V, is_, D):
    z = jnp.zeros((NPS, D), jnp.float32)
    out = _sc_segsum_linear_fn(D)(V, is_, z)
    return out.reshape(2, NPAD, D)


@functools.lru_cache(maxsize=None)
def _sc_hist_fn(D):
    @functools.partial(
        pl.kernel,
        out_type=_f32((2 * NPAD, D)),
        mesh=_mesh(),
        compiler_params=pltpu.CompilerParams(use_tc_tiling_on_sc=False),
        scratch_types=[
            pltpu.VMEM((128,), jnp.int32),
            pltpu.VMEM((128, D), jnp.float32),
            pltpu.VMEM_SHARED((NPAD, D), jnp.float32),
        ],
    )
    def k(ones_hbm, is_hbm, z_hbm, out_hbm, is_v, buf, acc):
        c = lax.axis_index("c")
        s = lax.axis_index("s")
        wid = s * 2 + c
        pltpu.sync_copy(z_hbm, acc.at[pl.ds(s * NPS, NPS)])
        pltpu.sync_copy(ones_hbm, buf)
        plsc.subcore_barrier()
        base = wid * EW

        def body(j, carry):
            off = base + j * 128
            pltpu.sync_copy(is_hbm.at[pl.ds(off, 128)], is_v)
            pltpu.sync_copy(buf, acc.at[is_v], add=True)
            return carry

        lax.fori_loop(0, NCH, body, 0)
        plsc.subcore_barrier()
        pltpu.sync_copy(acc.at[pl.ds(s * NPS, NPS)],
                        out_hbm.at[pl.ds(c * NPAD + s * NPS, NPS)])

    return k


def _sc_hist(is_, D):
    ones = jnp.ones((128, D), jnp.float32)
    z = jnp.zeros((NPS, D), jnp.float32)
    out = _sc_hist_fn(D)(ones, is_, z)
    return out.reshape(2, NPAD, D)


@functools.lru_cache(maxsize=None)
def _sc_gather_fn(D):
    @functools.partial(
        pl.kernel,
        out_type=_f32((E_PAD, D)),
        mesh=_mesh(),
        compiler_params=pltpu.CompilerParams(use_tc_tiling_on_sc=False),
        scratch_types=[
            pltpu.VMEM((128,), jnp.int32),
            pltpu.VMEM((128, D), jnp.float32),
            pltpu.SemaphoreType.DMA,
        ],
    )
    def k(u_hbm, ig_hbm, out_hbm, ig_v, buf, sem):
        c = lax.axis_index("c")
        s = lax.axis_index("s")
        wid = s * 2 + c
        base = wid * EW

        def body(j, carry):
            off = base + j * 128
            pltpu.sync_copy(ig_hbm.at[pl.ds(off, 128)], ig_v)
            pltpu.async_copy(u_hbm.at[ig_v], buf, sem).wait()
            pltpu.sync_copy(buf, out_hbm.at[pl.ds(off, 128)])
            return carry

        lax.fori_loop(0, NCH, body, 0)

    return k


def _sc_gather(U, ig, D):
    return _sc_gather_fn(D)(U, ig)


def _tc_input(x, w, b, g, b2):
    def body(x_r, w_r, b_r, g_r, b2_r, o_r):
        h = _gelu(jnp.dot(x_r[...], w_r[...],
                          preferred_element_type=jnp.float32) + b_r[...])
        o_r[...] = h * g_r[...] + b2_r[...]

    return pl.pallas_call(
        body,
        grid=(GBN,),
        in_specs=[_rows(RBN, IC), _full((IC, H)), _full((1, H)),
                  _full((1, H)), _full((1, H))],
        out_specs=_rows(RBN, H),
        out_shape=_f32((N, H)),
    )(x, w, b, g, b2)


def _tc_degdis(degp, cntp):
    def body(d_r, c_r, o_r):
        deg = d_r[0, :, 0:1] + d_r[1, :, 0:1]
        cnt = c_r[0, :, 0:1] + c_r[1, :, 0:1]
        dis = jnp.where(deg > 0, lax.rsqrt(jnp.maximum(deg, 1.0)), 0.0)
        cnt1 = jnp.maximum(cnt, 1.0)
        o_r[...] = jnp.concatenate(
            [dis, cnt1, jnp.zeros((dis.shape[0], 14), jnp.float32)], axis=-1)

    return pl.pallas_call(
        body,
        grid=(GBN,),
        in_specs=[_part(RBN, 16), _part(RBN, 16)],
        out_specs=_rows(RBN, 16),
        out_shape=_f32((N, 16)),
    )(degp, cntp)


def _tc_blockpre(h, dc, wcat, bcat):
    def body(h_r, dc_r, w_r, b_r, us_r, mid_r, p1_r, p2_r, q_r, kv_r):
        r = jnp.dot(h_r[...], w_r[...],
                    preferred_element_type=jnp.float32) + b_r[...]
        dis = dc_r[:, 0:1]
        us_r[...] = r[:, 0:32]
        mid_r[...] = jnp.concatenate(
            [r[:, 32:64], r[:, 128:160], r[:, 256:288]], axis=-1)
        p1_r[...] = dis * r[:, 64:96]
        p2_r[...] = dis * r[:, 96:128]
        q_r[...] = r[:, 160:192]
        kv_r[...] = r[:, 192:256]

    return pl.pallas_call(
        body,
        grid=(GBN,),
        in_specs=[_rows(RBN, H), _rows(RBN, 16), _full((H, 288)),
                  _full((1, 288))],
        out_specs=[_rows(RBN, 32), _rows(RBN, 96), _rows(RBN, 32),
                   _rows(RBN, 32), _rows(RBN, 32), _rows(RBN, 64)],
        out_shape=[_f32((N, 32)), _f32((N, 96)), _f32((N, 32)),
                   _f32((N, 32)), _f32((N, 32)), _f32((N, 64))],
    )(h, dc, wcat, bcat)


def _tc_blockmid(ssp, s1p, s2p, mid, dc):
    def body(ss_r, s1_r, s2_r, mid_r, dc_r, xc_r, p2c_r):
        ss = ss_r[0] + ss_r[1]
        s1 = s1_r[0] + s1_r[1]
        s2 = s2_r[0] + s2_r[1]
        dis = dc_r[:, 0:1]
        cnt1 = dc_r[:, 1:2]
        xl = _gelu(ss / cnt1 + mid_r[:, 0:32])
        c1 = mid_r[:, 32:64] - dis * s1
        xc_r[...] = jnp.concatenate([xl, c1], axis=-1)
        p2c_r[...] = -(dis * dis) * s2

    return pl.pallas_call(
        body,
        grid=(GBN,),
        in_specs=[_part(RBN, 32), _part(RBN, 32), _part(RBN, 32),
                  _rows(RBN, 96), _rows(RBN, 16)],
        out_specs=[_rows(RBN, 64), _rows(RBN, 32)],
        out_shape=[_f32((N, 64)), _f32((N, 32))],
    )(ssp, s1p, s2p, mid, dc)


def _tc_logit_max(qe, kve):
    def body(q_r, kv_r, m_r):
        i = pl.program_id(0)
        l = jnp.sum(q_r[...] * kv_r[:, 0:32], axis=-1,
                    keepdims=True) * (1.0 / SQRT_B)
        ids = i * RBE + lax.broadcasted_iota(jnp.int32, (RBE, 1), 0)
        l = jnp.where(ids < E, l, -1e30)
        m_r[...] = jnp.full((1, 1, 128), jnp.max(l), jnp.float32)

    return pl.pallas_call(
        body,
        grid=(GBE,),
        in_specs=[_rows(RBE, 32), _rows(RBE, 64)],
        out_specs=pl.BlockSpec((1, 1, 128), lambda i: (i, 0, 0)),
        out_shape=_f32((GBE, 1, 128)),
    )(qe, kve)


def _tc_expv(qe, kve, marr):
    def body(q_r, kv_r, m_r, ov_r, ox_r):
        i = pl.program_id(0)
        l = jnp.sum(q_r[...] * kv_r[:, 0:32], axis=-1,
                    keepdims=True) * (1.0 / SQRT_B)
        ids = i * RBE + lax.broadcasted_iota(jnp.int32, (RBE, 1), 0)
        ex = jnp.where(ids < E, jnp.exp(l - m_r[0, 0]), 0.0)
        ov_r[...] = kv_r[:, 32:64] * ex
        ox_r[...] = jnp.concatenate(
            [ex, jnp.zeros((RBE, 15), jnp.float32)], axis=-1)

    return pl.pallas_call(
        body,
        grid=(GBE,),
        in_specs=[_rows(RBE, 32), _rows(RBE, 64), _smem((1, 1))],
        out_specs=[_rows(RBE, 32), _rows(RBE, 16)],
        out_shape=[_f32((E_PAD, 32)), _f32((E_PAD, 16))],
    )(qe, kve, marr)


def _tc_blockpost(h, xc, s2bp, svp, sxp, mid, dc, wg, bg, lng, lnb):
    def body(h_r, xc_r, s2b_r, sv_r, sx_r, mid_r, dc_r, wg_r, bg_r, lng_r,
             lnb_r, o_r):
        dis = dc_r[:, 0:1]
        s2b = s2b_r[0] + s2b_r[1]
        sv = sv_r[0] + sv_r[1]
        sx = sx_r[0] + sx_r[1]
        xs = _gelu(xc_r[:, 32:64] - 2.0 * dis * s2b)
        att = sv / (sx[:, 0:1] + 1e-16)
        xa = _gelu(att + mid_r[:, 64:96])
        cat = jnp.concatenate([xc_r[:, 0:32], xs, xa], axis=-1)
        g = jax.nn.sigmoid(jnp.dot(cat, wg_r[...],
                                   preferred_element_type=jnp.float32)
                           + bg_r[...])
        o = g * cat + h_r[...]
        m = jnp.mean(o, axis=-1, keepdims=True)
        v = jnp.mean((o - m) ** 2, axis=-1, keepdims=True)
        o_r[...] = _gelu((o - m) / jnp.sqrt(v + 1e-5) * lng_r[...]
                         + lnb_r[...])

    return pl.pallas_call(
        body,
        grid=(GBN,),
        in_specs=[_rows(RBN, H), _rows(RBN, 64), _part(RBN, 32),
                  _part(RBN, 32), _part(RBN, 16), _rows(RBN, 96),
                  _rows(RBN, 16), _full((H, H)), _full((1, H)),
                  _full((1, H)), _full((1, H))],
        out_specs=_rows(RBN, H),
        out_shape=_f32((N, H)),
    )(h, xc, s2bp, svp, sxp, mid, dc, wg, bg, lng, lnb)


def _ln_in(x, g, b):
    m = jnp.mean(x, axis=-1, keepdims=True)
    v = jnp.mean((x - m) ** 2, axis=-1, keepdims=True)
    return (x - m) / jnp.sqrt(v + 1e-5) * g + b


def _tc_trans(h, w1, b1, w2, b2, w21, b21, w22, b22, sc1, sh1, ng1, nb1,
              sc2, sh2, ng2, nb2, tg, gatw, acat):
    def body(h_r, w1_r, b1_r, w2_r, b2_r, w21_r, b21_r, w22_r, b22_r,
             sc1_r, sh1_r, ng1_r, nb1_r, sc2_r, sh2_r, ng2_r, nb2_r,
             tg_r, gatw_r, acat_r, ht_r, xw_r, asd_r, ms_r):
        h = h_r[...]
        xt = h * sc1_r[...] + sh1_r[...]
        y = jnp.dot(_gelu(jnp.dot(xt, w1_r[...],
                                  preferred_element_type=jnp.float32)
                          + b1_r[...]), w2_r[...],
                    preferred_element_type=jnp.float32) + b2_r[...] + xt
        h = h + tg_r[0, 0] * _ln_in(y, ng1_r[...], nb1_r[...])
        xt2 = h * sc2_r[...] + sh2_r[...]
        y2 = jnp.dot(_gelu(jnp.dot(xt2, w21_r[...],
                                   preferred_element_type=jnp.float32)
                           + b21_r[...]), w22_r[...],
                     preferred_element_type=jnp.float32) + b22_r[...] + xt2
        h = h + tg_r[0, 1] * _ln_in(y2, ng2_r[...], nb2_r[...])
        ht_r[...] = h
        xw = jnp.dot(h, gatw_r[...], preferred_element_type=jnp.float32)
        xw_r[...] = xw
        asd8 = jnp.dot(xw, acat_r[...], preferred_element_type=jnp.float32)
        eself = asd8[:, 0:4] + asd8[:, 4:8]
        eself = jnp.where(eself > 0, eself, 0.2 * eself)
        asd_r[...] = jnp.concatenate(
            [asd8[:, 0:8], eself, jnp.zeros((RBN, 4), jnp.float32)], axis=-1)
        m4 = jnp.max(eself, axis=0, keepdims=True)
        ms_r[...] = jnp.concatenate(
            [m4, jnp.full((1, 124), -1e30, jnp.float32)],
            axis=-1).reshape(1, 1, 128)

    return pl.pallas_call(
        body,
        grid=(GBN,),
        in_specs=[_rows(RBN, H), _full((H, 2 * H)), _full((1, 2 * H)),
                  _full((2 * H, H)), _full((1, H)), _full((H, 2 * H)),
                  _full((1, 2 * H)), _full((2 * H, H)), _full((1, H)),
                  _full((1, H)), _full((1, H)), _full((1, H)), _full((1, H)),
                  _full((1, H)), _full((1, H)), _full((1, H)), _full((1, H)),
                  _smem((1, 2)), _full((H, H)), _full((H, 16))],
        out_specs=[_rows(RBN, H), _rows(RBN, H), _rows(RBN, 16),
                   pl.BlockSpec((1, 1, 128), lambda i: (i, 0, 0))],
        out_shape=[_f32((N, H)), _f32((N, H)), _f32((N, 16)),
                   _f32((GBN, 1, 128))],
    )(h, w1, b1, w2, b2, w21, b21, w22, b22, sc1, sh1, ng1, nb1,
      sc2, sh2, ng2, nb2, tg, gatw, acat)


def _tc_gat_emax(asr, asc):
    def body(r_r, c_r, m_r):
        i = pl.program_id(0)
        e = r_r[:, 0:4] + c_r[:, 4:8]
        e = jnp.where(e > 0, e, 0.2 * e)
        ids = i * RBE + lax.broadcasted_iota(jnp.int32, (RBE, 1), 0)
        e = jnp.where(ids < E, e, -1e30)
        m4 = jnp.max(e, axis=0, keepdims=True)
        m_r[...] = jnp.concatenate(
            [m4, jnp.full((1, 124), -1e30, jnp.float32)],
            axis=-1).reshape(1, 1, 128)

    return pl.pallas_call(
        body,
        grid=(GBE,),
        in_specs=[_rows(RBE, 16), _rows(RBE, 16)],
        out_specs=pl.BlockSpec((1, 1, 128), lambda i: (i, 0, 0)),
        out_shape=_f32((GBE, 1, 128)),
    )(asr, asc)


def _tc_gat_ex(asr, asc, m4):
    def body(r_r, c_r, m_r, o_r):
        i = pl.program_id(0)
        e = r_r[:, 0:4] + c_r[:, 4:8]
        e = jnp.where(e > 0, e, 0.2 * e)
        ids = i * RBE + lax.broadcasted_iota(jnp.int32, (RBE, 1), 0)
        ex = jnp.where(ids < E, jnp.exp(e - m_r[...]), 0.0)
        o_r[...] = jnp.concatenate(
            [ex, jnp.zeros((RBE, 12), jnp.float32)], axis=-1)

    return pl.pallas_call(
        body,
        grid=(GBE,),
        in_specs=[_rows(RBE, 16), _rows(RBE, 16), _full((1, 4))],
        out_specs=_rows(RBE, 16),
        out_shape=_f32((E_PAD, 16)),
    )(asr, asc, m4)


def _tc_gat_s(sp, asd, m4):
    def body(s_r, asd_r, m_r, o_r):
        s = s_r[0, :, 0:4] + s_r[1, :, 0:4]
        ex_self = jnp.exp(asd_r[:, 8:12] - m_r[...])
        s_tot = s + ex_self
        o_r[...] = jnp.concatenate(
            [s_tot, ex_self, jnp.zeros((RBN, 8), jnp.float32)], axis=-1)

    return pl.pallas_call(
        body,
        grid=(GBN,),
        in_specs=[_part(RBN, 16), _rows(RBN, 16), _full((1, 4))],
        out_specs=_rows(RBN, 16),
        out_shape=_f32((N, 16)),
    )(sp, asd, m4)


def _tc_gat_scale(ge, exe):
    def body(g_r, ex_r, o0_r, o1_r, o2_r):
        w4 = ex_r[:, 0:4]
        w96 = jnp.concatenate(
            [jnp.broadcast_to(w4[:, i:i + 1], (RBE2, DH)) for i in range(4)],
            axis=-1)
        gs = g_r[...] * w96
        o0_r[...] = gs[:, 0:32]
        o1_r[...] = gs[:, 32:64]
        o2_r[...] = gs[:, 64:96]

    return pl.pallas_call(
        body,
        grid=(GBE2,),
        in_specs=[_rows(RBE2, 96), _rows(RBE2, 16)],
        out_specs=[_rows(RBE2, 32), _rows(RBE2, 32), _rows(RBE2, 32)],
        out_shape=[_f32((E_PAD, 32)), _f32((E_PAD, 32)), _f32((E_PAD, 32))],
    )(ge, exe)


def _tc_final(h, xw, g0, g1, g2, sw, raw, gatb, bng, bnb,
              f1w, f1b, fng, fnb, f2w, f2b,
              gh1w, gh1b, gh2w, gh2b, gh3w, gh3b,
              fh1w, fh1b, fh2w, fh2b, mh1w, mh1b, mh2w, mh2b, skw, skb):
    def body(h_r, xw_r, g0_r, g1_r, g2_r, sw_r, raw_r, gatb_r, bng_r, bnb_r,
             f1w_r, f1b_r, fng_r, fnb_r, f2w_r, f2b_r,
             gh1w_r, gh1b_r, gh2w_r, gh2b_r, gh3w_r, gh3b_r,
             fh1w_r, fh1b_r, fh2w_r, fh2b_r,
             mh1w_r, mh1b_r, mh2w_r, mh2b_r, skw_r, skb_r, o_r):
        gagg = jnp.concatenate(
            [g0_r[0] + g0_r[1], g1_r[0] + g1_r[1], g2_r[0] + g2_r[1]],
            axis=-1)
        s_tot = sw_r[:, 0:4]
        ex_self = sw_r[:, 4:8]
        s96 = jnp.concatenate(
            [jnp.broadcast_to(s_tot[:, i:i + 1] + 1e-16, (RBN, DH))
             for i in range(4)], axis=-1)
        es96 = jnp.concatenate(
            [jnp.broadcast_to(ex_self[:, i:i + 1], (RBN, DH))
             for i in range(4)], axis=-1)
        gat = (gagg + xw_r[...] * es96) / s96 + gatb_r[...]
        h2 = h_r[...] + _gelu(gat * bng_r[...] + bnb_r[...])
        raw = raw_r[...]
        feat = _ln_in(_gelu(jnp.dot(raw, f1w_r[...],
                                    preferred_element_type=jnp.float32)
                            + f1b_r[...]), fng_r[...], fnb_r[...])
        feat = _gelu(jnp.dot(feat, f2w_r[...],
                             preferred_element_type=jnp.float32) + f2b_r[...])
        gp = _gelu(jnp.dot(h2, gh1w_r[...],
                           preferred_element_type=jnp.float32) + gh1b_r[...])
        gp = _gelu(jnp.dot(gp, gh2w_r[...],
                           preferred_element_type=jnp.float32) + gh2b_r[...])
        gp = jnp.dot(gp, gh3w_r[...],
                     preferred_element_type=jnp.float32) + gh3b_r[...]
        fp = jnp.dot(_gelu(jnp.dot(feat, fh1w_r[...],
                                   preferred_element_type=jnp.float32)
                           + fh1b_r[...]), fh2w_r[...],
                     preferred_element_type=jnp.float32) + fh2b_r[...]
        mix_in = jnp.concatenate([h2, feat], axis=-1)
        mix = jax.nn.sigmoid(
            jnp.dot(_gelu(jnp.dot(mix_in, mh1w_r[...],
                                  preferred_element_type=jnp.float32)
                          + mh1b_r[...]), mh2w_r[...],
                    preferred_element_type=jnp.float32) + mh2b_r[...])
        o_r[...] = (mix * gp + (1.0 - mix) * fp
                    + jnp.dot(raw, skw_r[...],
                              preferred_element_type=jnp.float32)
                    + skb_r[...])

    return pl.pallas_call(
        body,
        grid=(GBN,),
        in_specs=[_rows(RBN, H), _rows(RBN, H), _part(RBN, 32),
                  _part(RBN, 32), _part(RBN, 32), _rows(RBN, 16),
                  _rows(RBN, IC), _full((1, H)), _full((1, H)), _full((1, H)),
                  _full((IC, H)), _full((1, H)), _full((1, H)), _full((1, H)),
                  _full((H, H)), _full((1, H)),
                  _full((H, H)), _full((1, H)), _full((H, 48)),
                  _full((1, 48)), _full((48, OC)), _full((1, OC)),
                  _full((H, 48)), _full((1, 48)), _full((48, OC)),
                  _full((1, OC)), _full((2 * H, H)), _full((1, H)),
                  _full((H, OC)), _full((1, OC)), _full((IC, OC)),
                  _full((1, OC))],
        out_specs=_rows(RBN, OC),
        out_shape=_f32((N, OC)),
    )(h, xw, g0, g1, g2, sw, raw, gatb, bng, bnb,
      f1w, f1b, fng, fnb, f2w, f2b, gh1w, gh1b, gh2w, gh2b, gh3w, gh3b,
      fh1w, fh1b, fh2w, fh2b, mh1w, mh1b, mh2w, mh2b, skw, skb)


def _r1(v):
    return v.reshape(1, -1)


def _block(h, dc, row_p, col_p, col_ps, p):
    wcat = jnp.concatenate(
        [p["sage_l"]["w"], p["sage_r"], p["cheb_w"][1], p["cheb_w"][2],
         p["cheb_w"][0] - p["cheb_w"][2], p["tq"]["w"],
         p["tk"]["w"], p["tv"]["w"], p["tskip"]["w"]], axis=1)
    z32 = jnp.zeros((32,), jnp.float32)
    bcat = jnp.concatenate(
        [z32, p["sage_l"]["b"], z32, z32, p["cheb_b"],
         p["tq"]["b"] / SQRT_B, p["tk"]["b"], p["tv"]["b"],
         p["tskip"]["b"]]).reshape(1, 288)
    us, mid, p1, p2, q, kv = _tc_blockpre(h, dc, wcat, bcat)
    ssp = _sc_segsum_gather(us, row_p, col_ps, 32)
    s1p = _sc_segsum_gather(p1, row_p, col_ps, 32)
    s2p = _sc_segsum_gather(p2, row_p, col_ps, 32)
    qe = _sc_gather(q, col_p, 32)
    kve = _sc_gather(kv, row_p, 64)
    xc, p2c = _tc_blockmid(ssp, s1p, s2p, mid, dc)
    s2bp = _sc_segsum_gather(p2c, row_p, col_ps, 32)
    mx = _tc_logit_max(qe, kve)
    marr = jnp.max(mx).reshape(1, 1)
    vex, ex16 = _tc_expv(qe, kve, marr)
    svp = _sc_segsum_linear(vex, col_ps, 32)
    sxp = _sc_segsum_linear(ex16, col_ps, 16)
    return _tc_blockpost(h, xc, s2bp, svp, sxp, mid, dc, p["gate"]["w"],
                         _r1(p["gate"]["b"]), _r1(p["ln_g"]),
                         _r1(p["ln_b"]))


def kernel(x, edge_index, params):
    p = params
    row = edge_index[0]
    col = edge_index[1]
    npad_e = E_PAD - E
    padi = (jnp.arange(npad_e, dtype=jnp.int32) % 128)
    row_p = jnp.concatenate([row, padi])
    col_p = jnp.concatenate([col, padi])
    row_ps = jnp.concatenate([row, N + padi])
    col_ps = jnp.concatenate([col, N + padi])

    h = _tc_input(x, p["inp"]["w"], _r1(p["inp"]["b"]),
                  _r1(p["bn_inp_g"]), _r1(p["bn_inp_b"]))
    degp = _sc_hist(row_ps, 16)
    cntp = _sc_hist(col_ps, 16)
    dc = _tc_degdis(degp, cntp)

    h = _block(h, dc, row_p, col_p, col_ps, p["b1"])
    h = _block(h, dc, row_p, col_p, col_ps, p["b2"])
    h = _block(h, dc, row_p, col_p, col_ps, p["b3"])

    src_blocks = []
    for hh in range(HEADS):
        colv = jnp.zeros((DH, 16), jnp.float32)
        colv = colv.at[:, hh].set(p["att_src"][hh])
        colv = colv.at[:, 4 + hh].set(p["att_dst"][hh])
        src_blocks.append(colv)
    acat = jnp.concatenate(src_blocks, axis=0)

    tg = jnp.stack([jnp.tanh(p["t_gate"]),
                    jnp.tanh(p["t_gate2"])]).reshape(1, 2)
    ht, xw, asd, msb = _tc_trans(
        h, p["t_ff1"]["w"], _r1(p["t_ff1"]["b"]), p["t_ff2"]["w"],
        _r1(p["t_ff2"]["b"]), p["t2_ff1"]["w"], _r1(p["t2_ff1"]["b"]),
        p["t2_ff2"]["w"], _r1(p["t2_ff2"]["b"]),
        _r1(p["t_scale"]), _r1(p["t_shift"]), _r1(p["t_ng"]),
        _r1(p["t_nb"]), _r1(p["t_scale2"]), _r1(p["t_shift2"]),
        _r1(p["t2_ng"]), _r1(p["t2_nb"]), tg, p["gat_w"], acat)

    asr = _sc_gather(asd, row_p, 16)
    asc = _sc_gather(asd, col_p, 16)
    meb = _tc_gat_emax(asr, asc)
    m4 = jnp.max(jnp.concatenate([meb, msb], axis=0), axis=(0, 1))[:4]
    m4 = m4.reshape(1, 4)
    exe = _tc_gat_ex(asr, asc, m4)
    sp = _sc_segsum_linear(exe, col_ps, 16)
    sw = _tc_gat_s(sp, asd, m4)
    ge = _sc_gather(xw, row_p, 96)
    gs0, gs1, gs2 = _tc_gat_scale(ge, exe)
    g0 = _sc_segsum_linear(gs0, col_ps, 32)
    g1 = _sc_segsum_linear(gs1, col_ps, 32)
    g2 = _sc_segsum_linear(gs2, col_ps, 32)

    return _tc_final(
        ht, xw, g0, g1, g2, sw, x, _r1(p["gat_b"]), _r1(p["bn_g"]),
        _r1(p["bn_b"]), p["feat1"]["w"], _r1(p["feat1"]["b"]),
        _r1(p["feat_ng"]), _r1(p["feat_nb"]), p["feat2"]["w"],
        _r1(p["feat2"]["b"]), p["gh1"]["w"], _r1(p["gh1"]["b"]),
        p["gh2"]["w"], _r1(p["gh2"]["b"]), p["gh3"]["w"], _r1(p["gh3"]["b"]),
        p["fh1"]["w"], _r1(p["fh1"]["b"]), p["fh2"]["w"], _r1(p["fh2"]["b"]),
        p["mh1"]["w"], _r1(p["mh1"]["b"]), p["mh2"]["w"], _r1(p["mh2"]["b"]),
        p["skip"]["w"], _r1(p["skip"]["b"]))


# speedup vs baseline: 15.8988x; 1.3581x over previous
import functools

import jax
import jax.numpy as jnp
import numpy as np
from jax import lax
from jax.experimental import pallas as pl
from jax.experimental.pallas import tpu as pltpu
from jax.experimental.pallas import tpu_sc as plsc

N = 50000
E = 800000
IC, H, OC = 8, 96, 2
BD = H // 3
HEADS, DH = 4, H // 4
SQRT_B = np.sqrt(BD).astype(np.float32)

NW = 32
E_PAD = 819200
EW = E_PAD // NW
NCH = EW // 128
NPAD = 50176
NPS = NPAD // 16

RBN = 2000
GBN = N // RBN
RBE = 8192
GBE = E_PAD // RBE
RBE2 = 4096
GBE2 = E_PAD // RBE2

@functools.lru_cache(maxsize=None)
def _mesh():
    return plsc.VectorSubcoreMesh(core_axis_name="c", subcore_axis_name="s")


_INV_SQRT2 = np.float32(1.0 / np.sqrt(2.0))


def _gelu(x):
    return 0.5 * x * (1.0 + lax.erf(x * _INV_SQRT2))


def _rows(rb, d):
    return pl.BlockSpec((rb, d), lambda i: (i, 0))


def _part(rb, d):
    return pl.BlockSpec((2, rb, d), lambda i: (0, i, 0))


def _full(shape):
    return pl.BlockSpec(shape, lambda i: tuple(0 for _ in shape))


def _smem(shape):
    return pl.BlockSpec(shape, lambda i: tuple(0 for _ in shape),
                        memory_space=pltpu.SMEM)


def _f32(shape):
    return jax.ShapeDtypeStruct(shape, jnp.float32)


@functools.lru_cache(maxsize=None)
def _sc_segsum_gather_fn(D):
    NB = 4
    SEG = 40
    @functools.partial(
        pl.kernel,
        out_type=_f32((2 * NPAD, D)),
        mesh=_mesh(),
        compiler_params=pltpu.CompilerParams(use_tc_tiling_on_sc=False),
        scratch_types=[
            pltpu.VMEM((SEG, 128), jnp.int32),
            pltpu.VMEM((SEG, 128), jnp.int32),
            pltpu.VMEM((NB, 128, D), jnp.float32),
            pltpu.VMEM_SHARED((NPAD, D), jnp.float32),
            pltpu.SemaphoreType.DMA((NB,)),
        ],
    )
    def k(u_hbm, ig_hbm, is_hbm, z_hbm, out_hbm, ig2, is2, bufs, acc, gsem):
        c = lax.axis_index("c")
        s = lax.axis_index("s")
        wid = s * 2 + c
        pltpu.sync_copy(z_hbm, acc.at[pl.ds(s * NPS, NPS)])
        plsc.subcore_barrier()
        for seg in range(NCH // SEG):
            base_r = wid * NCH + seg * SEG
            pltpu.sync_copy(ig_hbm.at[pl.ds(base_r, SEG)], ig2)
            pltpu.sync_copy(is_hbm.at[pl.ds(base_r, SEG)], is2)
            for b in range(NB):
                pltpu.async_copy(u_hbm.at[ig2.at[b]], bufs.at[b], gsem.at[b])

            def outer(t, carry):
                for b in range(NB):
                    j = t * NB + b
                    pltpu.make_async_copy(u_hbm.at[ig2.at[b]], bufs.at[b],
                                          gsem.at[b]).wait()
                    pltpu.sync_copy(bufs.at[b], acc.at[is2.at[j]], add=True)
                    pltpu.async_copy(u_hbm.at[ig2.at[j + NB]], bufs.at[b],
                                     gsem.at[b])
                return carry

            lax.fori_loop(0, SEG // NB - 1, outer, 0)
            for b in range(NB):
                j = SEG - NB + b
                pltpu.make_async_copy(u_hbm.at[ig2.at[b]], bufs.at[b],
                                      gsem.at[b]).wait()
                pltpu.sync_copy(bufs.at[b], acc.at[is2.at[j]], add=True)
        plsc.subcore_barrier()
        pltpu.sync_copy(acc.at[pl.ds(s * NPS, NPS)],
                        out_hbm.at[pl.ds(c * NPAD + s * NPS, NPS)])

    return k


def _sc_segsum_gather(U, ig, is_, D):
    z = jnp.zeros((NPS, D), jnp.float32)
    out = _sc_segsum_gather_fn(D)(U, ig, is_, z)
    return out.reshape(2, NPAD, D)


@functools.lru_cache(maxsize=None)
def _sc_segsum_linear_fn(D):
    NB = 4
    SEG = 40
    @functools.partial(
        pl.kernel,
        out_type=_f32((2 * NPAD, D)),
        mesh=_mesh(),
        compiler_params=pltpu.CompilerParams(use_tc_tiling_on_sc=False),
        scratch_types=[
            pltpu.VMEM((SEG, 128), jnp.int32),
            pltpu.VMEM((NB, 128, D), jnp.float32),
            pltpu.VMEM_SHARED((NPAD, D), jnp.float32),
            pltpu.SemaphoreType.DMA((NB,)),
        ],
    )
    def k(v_hbm, is_hbm, z_hbm, out_hbm, is2, bufs, acc, gsem):
        c = lax.axis_index("c")
        s = lax.axis_index("s")
        wid = s * 2 + c
        pltpu.sync_copy(z_hbm, acc.at[pl.ds(s * NPS, NPS)])
        plsc.subcore_barrier()
        for seg in range(NCH // SEG):
            base_r = wid * NCH + seg * SEG
            pltpu.sync_copy(is_hbm.at[pl.ds(base_r, SEG)], is2)
            base = (wid * NCH + seg * SEG) * 128
            for b in range(NB):
                pltpu.async_copy(v_hbm.at[pl.ds(base + b * 128, 128)],
                                 bufs.at[b], gsem.at[b])

            def outer(t, carry):
                for b in range(NB):
                    j = t * NB + b
                    pltpu.make_async_copy(v_hbm.at[pl.ds(base, 128)],
                                          bufs.at[b], gsem.at[b]).wait()
                    pltpu.sync_copy(bufs.at[b], acc.at[is2.at[j]], add=True)
                    pltpu.async_copy(
                        v_hbm.at[pl.ds(base + (j + NB) * 128, 128)],
                        bufs.at[b], gsem.at[b])
                return carry

            lax.fori_loop(0, SEG // NB - 1, outer, 0)
            for b in range(NB):
                j = SEG - NB + b
                pltpu.make_async_copy(v_hbm.at[pl.ds(base, 128)], bufs.at[b],
                                      gsem.at[b]).wait()
                pltpu.sync_copy(bufs.at[b], acc.at[is2.at[j]], add=True)
        plsc.subcore_barrier()
        pltpu.sync_copy(acc.at[pl.ds(s * NPS, NPS)],
                        out_hbm.at[pl.ds(c * NPAD + s * NPS, NPS)])

    return k


def _sc_segsum_linear(V, is_, D):
    z = jnp.zeros((NPS, D), jnp.float32)
    out = _sc_segsum_linear_fn(D)(V, is_, z)
    return out.reshape(2, NPAD, D)


@functools.lru_cache(maxsize=None)
def _sc_hist_fn(D):
    @functools.partial(
        pl.kernel,
        out_type=_f32((2 * NPAD, D)),
        mesh=_mesh(),
        compiler_params=pltpu.CompilerParams(use_tc_tiling_on_sc=False),
        scratch_types=[
            pltpu.VMEM((NCH, 128), jnp.int32),
            pltpu.VMEM((128, D), jnp.float32),
            pltpu.VMEM_SHARED((NPAD, D), jnp.float32),
        ],
    )
    def k(ones_hbm, is_hbm, z_hbm, out_hbm, is2, buf, acc):
        c = lax.axis_index("c")
        s = lax.axis_index("s")
        wid = s * 2 + c
        pltpu.sync_copy(z_hbm, acc.at[pl.ds(s * NPS, NPS)])
        pltpu.sync_copy(ones_hbm, buf)
        base_r = wid * NCH
        pltpu.sync_copy(is_hbm.at[pl.ds(base_r, NCH)], is2)
        plsc.subcore_barrier()

        def body(j, carry):
            pltpu.sync_copy(buf, acc.at[is2.at[j]], add=True)
            return carry

        lax.fori_loop(0, NCH, body, 0)
        plsc.subcore_barrier()
        pltpu.sync_copy(acc.at[pl.ds(s * NPS, NPS)],
                        out_hbm.at[pl.ds(c * NPAD + s * NPS, NPS)])

    return k


def _sc_hist(is_, D):
    ones = jnp.ones((128, D), jnp.float32)
    z = jnp.zeros((NPS, D), jnp.float32)
    out = _sc_hist_fn(D)(ones, is_, z)
    return out.reshape(2, NPAD, D)


@functools.lru_cache(maxsize=None)
def _sc_gather_fn(D):
    NB = 8 if D <= 64 else 4
    @functools.partial(
        pl.kernel,
        out_type=_f32((E_PAD, D)),
        mesh=_mesh(),
        compiler_params=pltpu.CompilerParams(use_tc_tiling_on_sc=False),
        scratch_types=[
            pltpu.VMEM((NCH, 128), jnp.int32),
            pltpu.VMEM((NB, 128, D), jnp.float32),
            pltpu.SemaphoreType.DMA((NB,)),
        ],
    )
    def k(u_hbm, ig_hbm, out_hbm, ig2, bufs, gsem):
        c = lax.axis_index("c")
        s = lax.axis_index("s")
        wid = s * 2 + c
        base_r = wid * NCH
        pltpu.sync_copy(ig_hbm.at[pl.ds(base_r, NCH)], ig2)
        base = wid * EW
        for b in range(NB):
            pltpu.async_copy(u_hbm.at[ig2.at[b]], bufs.at[b], gsem.at[b])

        def outer(t, carry):
            for b in range(NB):
                j = t * NB + b
                pltpu.make_async_copy(u_hbm.at[ig2.at[b]], bufs.at[b],
                                      gsem.at[b]).wait()
                pltpu.sync_copy(bufs.at[b],
                                out_hbm.at[pl.ds(base + j * 128, 128)])
                pltpu.async_copy(u_hbm.at[ig2.at[j + NB]], bufs.at[b],
                                 gsem.at[b])
            return carry

        lax.fori_loop(0, NCH // NB - 1, outer, 0)
        for b in range(NB):
            j = NCH - NB + b
            pltpu.make_async_copy(u_hbm.at[ig2.at[b]], bufs.at[b],
                                  gsem.at[b]).wait()
            pltpu.sync_copy(bufs.at[b],
                            out_hbm.at[pl.ds(base + j * 128, 128)])

    return k


def _sc_gather(U, ig, D):
    return _sc_gather_fn(D)(U, ig)


def _tc_input(x, w, b, g, b2):
    def body(x_r, w_r, b_r, g_r, b2_r, o_r):
        h = _gelu(jnp.dot(x_r[...], w_r[...],
                          preferred_element_type=jnp.float32) + b_r[...])
        o_r[...] = h * g_r[...] + b2_r[...]

    return pl.pallas_call(
        body,
        grid=(GBN,),
        in_specs=[_rows(RBN, IC), _full((IC, H)), _full((1, H)),
                  _full((1, H)), _full((1, H))],
        out_specs=_rows(RBN, H),
        out_shape=_f32((N, H)),
    )(x, w, b, g, b2)


def _tc_degdis(degp, cntp):
    def body(d_r, c_r, o_r):
        deg = d_r[0, :, 0:1] + d_r[1, :, 0:1]
        cnt = c_r[0, :, 0:1] + c_r[1, :, 0:1]
        dis = jnp.where(deg > 0, lax.rsqrt(jnp.maximum(deg, 1.0)), 0.0)
        cnt1 = jnp.maximum(cnt, 1.0)
        o_r[...] = jnp.concatenate(
            [dis, cnt1, jnp.zeros((dis.shape[0], 14), jnp.float32)], axis=-1)

    return pl.pallas_call(
        body,
        grid=(GBN,),
        in_specs=[_part(RBN, 16), _part(RBN, 16)],
        out_specs=_rows(RBN, 16),
        out_shape=_f32((N, 16)),
    )(degp, cntp)


def _tc_blockpre(h, dc, wcat, bcat):
    def body(h_r, dc_r, w_r, b_r, us_r, mid_r, p1_r, p2_r, q_r, kv_r):
        r = jnp.dot(h_r[...], w_r[...],
                    preferred_element_type=jnp.float32) + b_r[...]
        dis = dc_r[:, 0:1]
        us_r[...] = r[:, 0:32]
        mid_r[...] = jnp.concatenate(
            [r[:, 32:64], r[:, 128:160], r[:, 256:288]], axis=-1)
        p1_r[...] = dis * r[:, 64:96]
        p2_r[...] = dis * r[:, 96:128]
        q_r[...] = r[:, 160:192]
        kv_r[...] = r[:, 192:256]

    return pl.pallas_call(
        body,
        grid=(GBN,),
        in_specs=[_rows(RBN, H), _rows(RBN, 16), _full((H, 288)),
                  _full((1, 288))],
        out_specs=[_rows(RBN, 32), _rows(RBN, 96), _rows(RBN, 32),
                   _rows(RBN, 32), _rows(RBN, 32), _rows(RBN, 64)],
        out_shape=[_f32((N, 32)), _f32((N, 96)), _f32((N, 32)),
                   _f32((N, 32)), _f32((N, 32)), _f32((N, 64))],
    )(h, dc, wcat, bcat)


def _tc_blockmid(ssp, s1p, s2p, mid, dc):
    def body(ss_r, s1_r, s2_r, mid_r, dc_r, xc_r, p2c_r):
        ss = ss_r[0] + ss_r[1]
        s1 = s1_r[0] + s1_r[1]
        s2 = s2_r[0] + s2_r[1]
        dis = dc_r[:, 0:1]
        cnt1 = dc_r[:, 1:2]
        xl = _gelu(ss / cnt1 + mid_r[:, 0:32])
        c1 = mid_r[:, 32:64] - dis * s1
        xc_r[...] = jnp.concatenate([xl, c1], axis=-1)
        p2c_r[...] = -(dis * dis) * s2

    return pl.pallas_call(
        body,
        grid=(GBN,),
        in_specs=[_part(RBN, 32), _part(RBN, 32), _part(RBN, 32),
                  _rows(RBN, 96), _rows(RBN, 16)],
        out_specs=[_rows(RBN, 64), _rows(RBN, 32)],
        out_shape=[_f32((N, 64)), _f32((N, 32))],
    )(ssp, s1p, s2p, mid, dc)


def _tc_logit_max(qe, kve):
    def body(q_r, kv_r, m_r):
        i = pl.program_id(0)
        l = jnp.sum(q_r[...] * kv_r[:, 0:32], axis=-1,
                    keepdims=True) * (1.0 / SQRT_B)
        ids = i * RBE + lax.broadcasted_iota(jnp.int32, (RBE, 1), 0)
        l = jnp.where(ids < E, l, -1e30)
        m_r[...] = jnp.full((1, 1, 128), jnp.max(l), jnp.float32)

    return pl.pallas_call(
        body,
        grid=(GBE,),
        in_specs=[_rows(RBE, 32), _rows(RBE, 64)],
        out_specs=pl.BlockSpec((1, 1, 128), lambda i: (i, 0, 0)),
        out_shape=_f32((GBE, 1, 128)),
    )(qe, kve)


def _tc_expv(qe, kve, marr):
    def body(q_r, kv_r, m_r, ov_r, ox_r):
        i = pl.program_id(0)
        l = jnp.sum(q_r[...] * kv_r[:, 0:32], axis=-1,
                    keepdims=True) * (1.0 / SQRT_B)
        ids = i * RBE + lax.broadcasted_iota(jnp.int32, (RBE, 1), 0)
        ex = jnp.where(ids < E, jnp.exp(l - m_r[0, 0]), 0.0)
        ov_r[...] = kv_r[:, 32:64] * ex
        ox_r[...] = jnp.concatenate(
            [ex, jnp.zeros((RBE, 15), jnp.float32)], axis=-1)

    return pl.pallas_call(
        body,
        grid=(GBE,),
        in_specs=[_rows(RBE, 32), _rows(RBE, 64), _smem((1, 1))],
        out_specs=[_rows(RBE, 32), _rows(RBE, 16)],
        out_shape=[_f32((E_PAD, 32)), _f32((E_PAD, 16))],
    )(qe, kve, marr)


def _tc_blockpost(h, xc, s2bp, svp, sxp, mid, dc, wg, bg, lng, lnb):
    def body(h_r, xc_r, s2b_r, sv_r, sx_r, mid_r, dc_r, wg_r, bg_r, lng_r,
             lnb_r, o_r):
        dis = dc_r[:, 0:1]
        s2b = s2b_r[0] + s2b_r[1]
        sv = sv_r[0] + sv_r[1]
        sx = sx_r[0] + sx_r[1]
        xs = _gelu(xc_r[:, 32:64] - 2.0 * dis * s2b)
        att = sv / (sx[:, 0:1] + 1e-16)
        xa = _gelu(att + mid_r[:, 64:96])
        cat = jnp.concatenate([xc_r[:, 0:32], xs, xa], axis=-1)
        g = jax.nn.sigmoid(jnp.dot(cat, wg_r[...],
                                   preferred_element_type=jnp.float32)
                           + bg_r[...])
        o = g * cat + h_r[...]
        m = jnp.mean(o, axis=-1, keepdims=True)
        v = jnp.mean((o - m) ** 2, axis=-1, keepdims=True)
        o_r[...] = _gelu((o - m) / jnp.sqrt(v + 1e-5) * lng_r[...]
                         + lnb_r[...])

    return pl.pallas_call(
        body,
        grid=(GBN,),
        in_specs=[_rows(RBN, H), _rows(RBN, 64), _part(RBN, 32),
                  _part(RBN, 32), _part(RBN, 16), _rows(RBN, 96),
                  _rows(RBN, 16), _full((H, H)), _full((1, H)),
                  _full((1, H)), _full((1, H))],
        out_specs=_rows(RBN, H),
        out_shape=_f32((N, H)),
    )(h, xc, s2bp, svp, sxp, mid, dc, wg, bg, lng, lnb)


def _ln_in(x, g, b):
    m = jnp.mean(x, axis=-1, keepdims=True)
    v = jnp.mean((x - m) ** 2, axis=-1, keepdims=True)
    return (x - m) / jnp.sqrt(v + 1e-5) * g + b


def _tc_trans(h, w1, b1, w2, b2, w21, b21, w22, b22, sc1, sh1, ng1, nb1,
              sc2, sh2, ng2, nb2, tg, gatw, acat):
    def body(h_r, w1_r, b1_r, w2_r, b2_r, w21_r, b21_r, w22_r, b22_r,
             sc1_r, sh1_r, ng1_r, nb1_r, sc2_r, sh2_r, ng2_r, nb2_r,
             tg_r, gatw_r, acat_r, ht_r, xw_r, asd_r, ms_r):
        h = h_r[...]
        xt = h * sc1_r[...] + sh1_r[...]
        y = jnp.dot(_gelu(jnp.dot(xt, w1_r[...],
                                  preferred_element_type=jnp.float32)
                          + b1_r[...]), w2_r[...],
                    preferred_element_type=jnp.float32) + b2_r[...] + xt
        h = h + tg_r[0, 0] * _ln_in(y, ng1_r[...], nb1_r[...])
        xt2 = h * sc2_r[...] + sh2_r[...]
        y2 = jnp.dot(_gelu(jnp.dot(xt2, w21_r[...],
                                   preferred_element_type=jnp.float32)
                           + b21_r[...]), w22_r[...],
                     preferred_element_type=jnp.float32) + b22_r[...] + xt2
        h = h + tg_r[0, 1] * _ln_in(y2, ng2_r[...], nb2_r[...])
        ht_r[...] = h
        xw = jnp.dot(h, gatw_r[...], preferred_element_type=jnp.float32)
        xw_r[...] = xw
        asd8 = jnp.dot(xw, acat_r[...], preferred_element_type=jnp.float32)
        eself = asd8[:, 0:4] + asd8[:, 4:8]
        eself = jnp.where(eself > 0, eself, 0.2 * eself)
        asd_r[...] = jnp.concatenate(
            [asd8[:, 0:8], eself, jnp.zeros((RBN, 4), jnp.float32)], axis=-1)
        m4 = jnp.max(eself, axis=0, keepdims=True)
        ms_r[...] = jnp.concatenate(
            [m4, jnp.full((1, 124), -1e30, jnp.float32)],
            axis=-1).reshape(1, 1, 128)

    return pl.pallas_call(
        body,
        grid=(GBN,),
        in_specs=[_rows(RBN, H), _full((H, 2 * H)), _full((1, 2 * H)),
                  _full((2 * H, H)), _full((1, H)), _full((H, 2 * H)),
                  _full((1, 2 * H)), _full((2 * H, H)), _full((1, H)),
                  _full((1, H)), _full((1, H)), _full((1, H)), _full((1, H)),
                  _full((1, H)), _full((1, H)), _full((1, H)), _full((1, H)),
                  _smem((1, 2)), _full((H, H)), _full((H, 16))],
        out_specs=[_rows(RBN, H), _rows(RBN, H), _rows(RBN, 16),
                   pl.BlockSpec((1, 1, 128), lambda i: (i, 0, 0))],
        out_shape=[_f32((N, H)), _f32((N, H)), _f32((N, 16)),
                   _f32((GBN, 1, 128))],
    )(h, w1, b1, w2, b2, w21, b21, w22, b22, sc1, sh1, ng1, nb1,
      sc2, sh2, ng2, nb2, tg, gatw, acat)


def _tc_gat_emax(asr, asc):
    def body(r_r, c_r, m_r):
        i = pl.program_id(0)
        e = r_r[:, 0:4] + c_r[:, 4:8]
        e = jnp.where(e > 0, e, 0.2 * e)
        ids = i * RBE + lax.broadcasted_iota(jnp.int32, (RBE, 1), 0)
        e = jnp.where(ids < E, e, -1e30)
        m4 = jnp.max(e, axis=0, keepdims=True)
        m_r[...] = jnp.concatenate(
            [m4, jnp.full((1, 124), -1e30, jnp.float32)],
            axis=-1).reshape(1, 1, 128)

    return pl.pallas_call(
        body,
        grid=(GBE,),
        in_specs=[_rows(RBE, 16), _rows(RBE, 16)],
        out_specs=pl.BlockSpec((1, 1, 128), lambda i: (i, 0, 0)),
        out_shape=_f32((GBE, 1, 128)),
    )(asr, asc)


def _tc_gat_ex(asr, asc, m4):
    def body(r_r, c_r, m_r, o_r):
        i = pl.program_id(0)
        e = r_r[:, 0:4] + c_r[:, 4:8]
        e = jnp.where(e > 0, e, 0.2 * e)
        ids = i * RBE + lax.broadcasted_iota(jnp.int32, (RBE, 1), 0)
        ex = jnp.where(ids < E, jnp.exp(e - m_r[...]), 0.0)
        o_r[...] = jnp.concatenate(
            [ex, jnp.zeros((RBE, 12), jnp.float32)], axis=-1)

    return pl.pallas_call(
        body,
        grid=(GBE,),
        in_specs=[_rows(RBE, 16), _rows(RBE, 16), _full((1, 4))],
        out_specs=_rows(RBE, 16),
        out_shape=_f32((E_PAD, 16)),
    )(asr, asc, m4)


def _tc_gat_s(sp, asd, m4):
    def body(s_r, asd_r, m_r, o_r):
        s = s_r[0, :, 0:4] + s_r[1, :, 0:4]
        ex_self = jnp.exp(asd_r[:, 8:12] - m_r[...])
        s_tot = s + ex_self
        o_r[...] = jnp.concatenate(
            [s_tot, ex_self, jnp.zeros((RBN, 8), jnp.float32)], axis=-1)

    return pl.pallas_call(
        body,
        grid=(GBN,),
        in_specs=[_part(RBN, 16), _rows(RBN, 16), _full((1, 4))],
        out_specs=_rows(RBN, 16),
        out_shape=_f32((N, 16)),
    )(sp, asd, m4)


def _tc_gat_scale(ge, exe):
    def body(g_r, ex_r, o0_r, o1_r, o2_r):
        w4 = ex_r[:, 0:4]
        w96 = jnp.concatenate(
            [jnp.broadcast_to(w4[:, i:i + 1], (RBE2, DH)) for i in range(4)],
            axis=-1)
        gs = g_r[...] * w96
        o0_r[...] = gs[:, 0:32]
        o1_r[...] = gs[:, 32:64]
        o2_r[...] = gs[:, 64:96]

    return pl.pallas_call(
        body,
        grid=(GBE2,),
        in_specs=[_rows(RBE2, 96), _rows(RBE2, 16)],
        out_specs=[_rows(RBE2, 32), _rows(RBE2, 32), _rows(RBE2, 32)],
        out_shape=[_f32((E_PAD, 32)), _f32((E_PAD, 32)), _f32((E_PAD, 32))],
    )(ge, exe)


def _tc_final(h, xw, g0, g1, g2, sw, raw, gatb, bng, bnb,
              f1w, f1b, fng, fnb, f2w, f2b,
              gh1w, gh1b, gh2w, gh2b, gh3w, gh3b,
              fh1w, fh1b, fh2w, fh2b, mh1w, mh1b, mh2w, mh2b, skw, skb):
    def body(h_r, xw_r, g0_r, g1_r, g2_r, sw_r, raw_r, gatb_r, bng_r, bnb_r,
             f1w_r, f1b_r, fng_r, fnb_r, f2w_r, f2b_r,
             gh1w_r, gh1b_r, gh2w_r, gh2b_r, gh3w_r, gh3b_r,
             fh1w_r, fh1b_r, fh2w_r, fh2b_r,
             mh1w_r, mh1b_r, mh2w_r, mh2b_r, skw_r, skb_r, o_r):
        gagg = jnp.concatenate(
            [g0_r[0] + g0_r[1], g1_r[0] + g1_r[1], g2_r[0] + g2_r[1]],
            axis=-1)
        s_tot = sw_r[:, 0:4]
        ex_self = sw_r[:, 4:8]
        s96 = jnp.concatenate(
            [jnp.broadcast_to(s_tot[:, i:i + 1] + 1e-16, (RBN, DH))
             for i in range(4)], axis=-1)
        es96 = jnp.concatenate(
            [jnp.broadcast_to(ex_self[:, i:i + 1], (RBN, DH))
             for i in range(4)], axis=-1)
        gat = (gagg + xw_r[...] * es96) / s96 + gatb_r[...]
        h2 = h_r[...] + _gelu(gat * bng_r[...] + bnb_r[...])
        raw = raw_r[...]
        feat = _ln_in(_gelu(jnp.dot(raw, f1w_r[...],
                                    preferred_element_type=jnp.float32)
                            + f1b_r[...]), fng_r[...], fnb_r[...])
        feat = _gelu(jnp.dot(feat, f2w_r[...],
                             preferred_element_type=jnp.float32) + f2b_r[...])
        gp = _gelu(jnp.dot(h2, gh1w_r[...],
                           preferred_element_type=jnp.float32) + gh1b_r[...])
        gp = _gelu(jnp.dot(gp, gh2w_r[...],
                           preferred_element_type=jnp.float32) + gh2b_r[...])
        gp = jnp.dot(gp, gh3w_r[...],
                     preferred_element_type=jnp.float32) + gh3b_r[...]
        fp = jnp.dot(_gelu(jnp.dot(feat, fh1w_r[...],
                                   preferred_element_type=jnp.float32)
                           + fh1b_r[...]), fh2w_r[...],
                     preferred_element_type=jnp.float32) + fh2b_r[...]
        mix_in = jnp.concatenate([h2, feat], axis=-1)
        mix = jax.nn.sigmoid(
            jnp.dot(_gelu(jnp.dot(mix_in, mh1w_r[...],
                                  preferred_element_type=jnp.float32)
                          + mh1b_r[...]), mh2w_r[...],
                    preferred_element_type=jnp.float32) + mh2b_r[...])
        o_r[...] = (mix * gp + (1.0 - mix) * fp
                    + jnp.dot(raw, skw_r[...],
                              preferred_element_type=jnp.float32)
                    + skb_r[...])

    return pl.pallas_call(
        body,
        grid=(GBN,),
        in_specs=[_rows(RBN, H), _rows(RBN, H), _part(RBN, 32),
                  _part(RBN, 32), _part(RBN, 32), _rows(RBN, 16),
                  _rows(RBN, IC), _full((1, H)), _full((1, H)), _full((1, H)),
                  _full((IC, H)), _full((1, H)), _full((1, H)), _full((1, H)),
                  _full((H, H)), _full((1, H)),
                  _full((H, H)), _full((1, H)), _full((H, 48)),
                  _full((1, 48)), _full((48, OC)), _full((1, OC)),
                  _full((H, 48)), _full((1, 48)), _full((48, OC)),
                  _full((1, OC)), _full((2 * H, H)), _full((1, H)),
                  _full((H, OC)), _full((1, OC)), _full((IC, OC)),
                  _full((1, OC))],
        out_specs=_rows(RBN, OC),
        out_shape=_f32((N, OC)),
    )(h, xw, g0, g1, g2, sw, raw, gatb, bng, bnb,
      f1w, f1b, fng, fnb, f2w, f2b, gh1w, gh1b, gh2w, gh2b, gh3w, gh3b,
      fh1w, fh1b, fh2w, fh2b, mh1w, mh1b, mh2w, mh2b, skw, skb)


def _r1(v):
    return v.reshape(1, -1)


def _block(h, dc, row_p, col_p, col_ps, p):
    wcat = jnp.concatenate(
        [p["sage_l"]["w"], p["sage_r"], p["cheb_w"][1], p["cheb_w"][2],
         p["cheb_w"][0] - p["cheb_w"][2], p["tq"]["w"],
         p["tk"]["w"], p["tv"]["w"], p["tskip"]["w"]], axis=1)
    z32 = jnp.zeros((32,), jnp.float32)
    bcat = jnp.concatenate(
        [z32, p["sage_l"]["b"], z32, z32, p["cheb_b"],
         p["tq"]["b"] / SQRT_B, p["tk"]["b"], p["tv"]["b"],
         p["tskip"]["b"]]).reshape(1, 288)
    us, mid, p1, p2, q, kv = _tc_blockpre(h, dc, wcat, bcat)
    ssp = _sc_segsum_gather(us, row_p, col_ps, 32)
    s1p = _sc_segsum_gather(p1, row_p, col_ps, 32)
    s2p = _sc_segsum_gather(p2, row_p, col_ps, 32)
    qe = _sc_gather(q, col_p, 32)
    kve = _sc_gather(kv, row_p, 64)
    xc, p2c = _tc_blockmid(ssp, s1p, s2p, mid, dc)
    s2bp = _sc_segsum_gather(p2c, row_p, col_ps, 32)
    mx = _tc_logit_max(qe, kve)
    marr = jnp.max(mx).reshape(1, 1)
    vex, ex16 = _tc_expv(qe, kve, marr)
    svp = _sc_segsum_linear(vex, col_ps, 32)
    sxp = _sc_segsum_linear(ex16, col_ps, 16)
    return _tc_blockpost(h, xc, s2bp, svp, sxp, mid, dc, p["gate"]["w"],
                         _r1(p["gate"]["b"]), _r1(p["ln_g"]),
                         _r1(p["ln_b"]))


def kernel(x, edge_index, params):
    p = params
    row = edge_index[0]
    col = edge_index[1]
    npad_e = E_PAD - E
    padi = (jnp.arange(npad_e, dtype=jnp.int32) % 128)
    row_p = jnp.concatenate([row, padi]).reshape(E_PAD // 128, 128)
    col_p = jnp.concatenate([col, padi]).reshape(E_PAD // 128, 128)
    row_ps = jnp.concatenate([row, N + padi]).reshape(E_PAD // 128, 128)
    col_ps = jnp.concatenate([col, N + padi]).reshape(E_PAD // 128, 128)

    h = _tc_input(x, p["inp"]["w"], _r1(p["inp"]["b"]),
                  _r1(p["bn_inp_g"]), _r1(p["bn_inp_b"]))
    degp = _sc_hist(row_ps, 16)
    cntp = _sc_hist(col_ps, 16)
    dc = _tc_degdis(degp, cntp)

    h = _block(h, dc, row_p, col_p, col_ps, p["b1"])
    h = _block(h, dc, row_p, col_p, col_ps, p["b2"])
    h = _block(h, dc, row_p, col_p, col_ps, p["b3"])

    src_blocks = []
    for hh in range(HEADS):
        colv = jnp.zeros((DH, 16), jnp.float32)
        colv = colv.at[:, hh].set(p["att_src"][hh])
        colv = colv.at[:, 4 + hh].set(p["att_dst"][hh])
        src_blocks.append(colv)
    acat = jnp.concatenate(src_blocks, axis=0)

    tg = jnp.stack([jnp.tanh(p["t_gate"]),
                    jnp.tanh(p["t_gate2"])]).reshape(1, 2)
    ht, xw, asd, msb = _tc_trans(
        h, p["t_ff1"]["w"], _r1(p["t_ff1"]["b"]), p["t_ff2"]["w"],
        _r1(p["t_ff2"]["b"]), p["t2_ff1"]["w"], _r1(p["t2_ff1"]["b"]),
        p["t2_ff2"]["w"], _r1(p["t2_ff2"]["b"]),
        _r1(p["t_scale"]), _r1(p["t_shift"]), _r1(p["t_ng"]),
        _r1(p["t_nb"]), _r1(p["t_scale2"]), _r1(p["t_shift2"]),
        _r1(p["t2_ng"]), _r1(p["t2_nb"]), tg, p["gat_w"], acat)

    asr = _sc_gather(asd, row_p, 16)
    asc = _sc_gather(asd, col_p, 16)
    meb = _tc_gat_emax(asr, asc)
    m4 = jnp.max(jnp.concatenate([meb, msb], axis=0), axis=(0, 1))[:4]
    m4 = m4.reshape(1, 4)
    exe = _tc_gat_ex(asr, asc, m4)
    sp = _sc_segsum_linear(exe, col_ps, 16)
    sw = _tc_gat_s(sp, asd, m4)
    ge = _sc_gather(xw, row_p, 96)
    gs0, gs1, gs2 = _tc_gat_scale(ge, exe)
    g0 = _sc_segsum_linear(gs0, col_ps, 32)
    g1 = _sc_segsum_linear(gs1, col_ps, 32)
    g2 = _sc_segsum_linear(gs2, col_ps, 32)

    return _tc_final(
        ht, xw, g0, g1, g2, sw, x, _r1(p["gat_b"]), _r1(p["bn_g"]),
        _r1(p["bn_b"]), p["feat1"]["w"], _r1(p["feat1"]["b"]),
        _r1(p["feat_ng"]), _r1(p["feat_nb"]), p["feat2"]["w"],
        _r1(p["feat2"]["b"]), p["gh1"]["w"], _r1(p["gh1"]["b"]),
        p["gh2"]["w"], _r1(p["gh2"]["b"]), p["gh3"]["w"], _r1(p["gh3"]["b"]),
        p["fh1"]["w"], _r1(p["fh1"]["b"]), p["fh2"]["w"], _r1(p["fh2"]["b"]),
        p["mh1"]["w"], _r1(p["mh1"]["b"]), p["mh2"]["w"], _r1(p["mh2"]["b"]),
        p["skip"]["w"], _r1(p["skip"]["b"]))
